# baseline (speedup 1.0000x reference)
import functools

import numpy as np
import jax
import jax.numpy as jnp
from jax import lax
from jax.experimental import pallas as pl
from jax.experimental.pallas import tpu as pltpu
from jax.experimental.pallas import tpu_sc as plsc

_PATCH = 16
_PROB = 0.0

_MASK576_HEX = (
    "8a222eb193a459cdd7668e1a933c91e44ca8c361a99a316ed8f9c3e88cb12d8b"
    "5884d418566c9ac96c3f9aafa0fe2bb9431b6aebd58ff313fcde0029f1c7a40c"
    "cb52128792169864"
)


@functools.lru_cache(maxsize=None)
def _keep_patch(nh: int, nw: int) -> np.ndarray:
    L = nh * nw
    if L == 576:
        mask = np.unpackbits(
            np.frombuffer(bytes.fromhex(_MASK576_HEX), dtype=np.uint8)
        )[:L].astype(bool)
    else:
        with jax.ensure_compile_time_eval():
            r = np.asarray(
                jax.random.normal(jax.random.key(42), (L,), dtype=jnp.float32)
            )
        mask = r < _PROB
    return (~mask).reshape(nh, nw).astype(np.float32)


def _keep_flat(nh: int, nw: int) -> np.ndarray:
    keep = _keep_patch(nh, nw)
    keep2d = np.repeat(np.repeat(keep, _PATCH, 0), _PATCH, 1)
    return keep2d.reshape(1, nh * _PATCH * nw * _PATCH)


def _mask_body(x_ref, m_ref, o_ref):
    o_ref[...] = x_ref[...] * m_ref[...]


def _kernel_tc(x):
    B, C, H, W = x.shape
    nh, nw = H // _PATCH, W // _PATCH
    keep = jnp.asarray(_keep_flat(nh, nw)).reshape(1, H, W)

    rows = B * C
    xf = x.reshape(rows, H, W)
    bm = 24
    out = pl.pallas_call(
        _mask_body,
        grid=(rows // bm,),
        in_specs=[
            pl.BlockSpec((bm, H, W), lambda i: (i, 0, 0)),
            pl.BlockSpec((1, H, W), lambda i: (0, 0, 0)),
        ],
        out_specs=pl.BlockSpec((bm, H, W), lambda i: (i, 0, 0)),
        out_shape=jax.ShapeDtypeStruct((rows, H, W), x.dtype),
    )(xf, keep)
    return out.reshape(B, C, H, W)


_NC, _NS = 2, 16
_NW = _NC * _NS


def _kernel_sc(x):
    B, C, H, W = x.shape
    p = _PATCH
    nh, nw = H // p, W // p
    rows = B * C
    imgs_per_w = rows // _NW
    bands_per_w = imgs_per_w * nh

    keep = _keep_patch(nh, nw)
    mrows = np.repeat(keep, p, axis=1)

    x3 = x.reshape(rows, H, W)

    mesh = plsc.VectorSubcoreMesh(core_axis_name="c", subcore_axis_name="s")

    @functools.partial(
        pl.kernel,
        mesh=mesh,
        out_type=jax.ShapeDtypeStruct((rows, H, W), x.dtype),
        scratch_types=[
            pltpu.VMEM((nh, W), jnp.float32),
            pltpu.VMEM((p, W), jnp.float32),
        ],
    )
    def sc_k(x_hbm, mrows_hbm, out_hbm, mrows_v, buf_v):
        wid = lax.axis_index("s") * _NC + lax.axis_index("c")
        pltpu.sync_copy(mrows_hbm, mrows_v)

        def band_body(t, carry):
            img = wid * imgs_per_w + t // nh
            brow = t % nh
            pltpu.sync_copy(x_hbm.at[img, pl.ds(brow * p, p), :], buf_v)
            mvs = [mrows_v[brow, pl.ds(c * 16, 16)] for c in range(W // 16)]
            for r in range(p):
                for c in range(W // 16):
                    sl = pl.ds(c * 16, 16)
                    buf_v[r, sl] = buf_v[r, sl] * mvs[c]
            pltpu.sync_copy(buf_v, out_hbm.at[img, pl.ds(brow * p, p), :])
            return carry

        lax.fori_loop(0, bands_per_w, band_body, 0)

    out = sc_k(x3, jnp.asarray(mrows))
    return out.reshape(B, C, H, W)


def kernel(x):
    return _kernel_sc(x)


# speedup vs baseline: 1.3757x; 1.3757x over previous
import functools

import numpy as np
import jax
import jax.numpy as jnp
from jax import lax
from jax.experimental import pallas as pl
from jax.experimental.pallas import tpu as pltpu
from jax.experimental.pallas import tpu_sc as plsc

_PATCH = 16
_PROB = 0.0

_MASK576_HEX = (
    "8a222eb193a459cdd7668e1a933c91e44ca8c361a99a316ed8f9c3e88cb12d8b"
    "5884d418566c9ac96c3f9aafa0fe2bb9431b6aebd58ff313fcde0029f1c7a40c"
    "cb52128792169864"
)


@functools.lru_cache(maxsize=None)
def _keep_patch(nh: int, nw: int) -> np.ndarray:
    L = nh * nw
    if L == 576:
        mask = np.unpackbits(
            np.frombuffer(bytes.fromhex(_MASK576_HEX), dtype=np.uint8)
        )[:L].astype(bool)
    else:
        with jax.ensure_compile_time_eval():
            r = np.asarray(
                jax.random.normal(jax.random.key(42), (L,), dtype=jnp.float32)
            )
        mask = r < _PROB
    return (~mask).reshape(nh, nw).astype(np.float32)


def _keep_flat(nh: int, nw: int) -> np.ndarray:
    keep = _keep_patch(nh, nw)
    keep2d = np.repeat(np.repeat(keep, _PATCH, 0), _PATCH, 1)
    return keep2d.reshape(1, nh * _PATCH * nw * _PATCH)


def _mask_body(x_ref, m_ref, o_ref):
    o_ref[...] = x_ref[...] * m_ref[...]


def _kernel_tc(x):
    B, C, H, W = x.shape
    nh, nw = H // _PATCH, W // _PATCH
    keep = jnp.asarray(_keep_flat(nh, nw)).reshape(1, H, W)

    rows = B * C
    xf = x.reshape(rows, H, W)
    bm = 24
    out = pl.pallas_call(
        _mask_body,
        grid=(rows // bm,),
        in_specs=[
            pl.BlockSpec((bm, H, W), lambda i: (i, 0, 0)),
            pl.BlockSpec((1, H, W), lambda i: (0, 0, 0)),
        ],
        out_specs=pl.BlockSpec((bm, H, W), lambda i: (i, 0, 0)),
        out_shape=jax.ShapeDtypeStruct((rows, H, W), x.dtype),
    )(xf, keep)
    return out.reshape(B, C, H, W)


_NC, _NS = 2, 16
_NW = _NC * _NS


def _kernel_sc(x):
    B, C, H, W = x.shape
    p = _PATCH
    nh, nw = H // p, W // p
    rows = B * C
    imgs_per_w = rows // _NW
    bands_per_w = imgs_per_w * nh

    keep = _keep_patch(nh, nw)
    mrows = np.repeat(keep, p, axis=1)

    x3 = x.reshape(rows, H, W)

    mesh = plsc.VectorSubcoreMesh(core_axis_name="c", subcore_axis_name="s")

    @functools.partial(
        pl.kernel,
        mesh=mesh,
        out_type=jax.ShapeDtypeStruct((rows, H, W), x.dtype),
        scratch_types=[
            pltpu.VMEM((nh, W), jnp.float32),
            pltpu.VMEM((p, W), jnp.float32),
        ],
    )
    def sc_k(x_hbm, mrows_hbm, out_hbm, mrows_v, buf_v):
        wid = lax.axis_index("s") * _NC + lax.axis_index("c")
        pltpu.sync_copy(mrows_hbm, mrows_v)

        def band_body(t, carry):
            img = wid * imgs_per_w + t // nh
            brow = t % nh
            pltpu.sync_copy(x_hbm.at[img, pl.ds(brow * p, p), :], buf_v)
            mvs = [mrows_v[brow, pl.ds(c * 16, 16)] for c in range(W // 16)]
            for r in range(p):
                for c in range(W // 16):
                    sl = pl.ds(c * 16, 16)
                    buf_v[r, sl] = buf_v[r, sl] * mvs[c]
            pltpu.sync_copy(buf_v, out_hbm.at[img, pl.ds(brow * p, p), :])
            return carry

        lax.fori_loop(0, bands_per_w, band_body, 0)

    out = sc_k(x3, jnp.asarray(mrows))
    return out.reshape(B, C, H, W)


def _kernel_hybrid(x, sc_rows=96):
    B, C, H, W = x.shape
    p = _PATCH
    nh, nw = H // p, W // p
    rows = B * C
    tc_rows = rows - sc_rows
    imgs_per_w = sc_rows // _NW
    bands_per_w = imgs_per_w * nh

    keep = _keep_patch(nh, nw)
    mrows = np.repeat(keep, p, axis=1)
    x3 = x.reshape(rows, H, W)

    mesh = plsc.VectorSubcoreMesh(core_axis_name="c", subcore_axis_name="s")

    @functools.partial(
        pl.kernel,
        mesh=mesh,
        out_type=jax.ShapeDtypeStruct((sc_rows, H, W), x.dtype),
        scratch_types=[
            pltpu.VMEM((nh, W), jnp.float32),
            pltpu.VMEM((p, W), jnp.float32),
        ],
    )
    def sc_k(x_hbm, mrows_hbm, out_hbm, mrows_v, buf_v):
        wid = lax.axis_index("s") * _NC + lax.axis_index("c")
        pltpu.sync_copy(mrows_hbm, mrows_v)

        def band_body(t, carry):
            img = wid * imgs_per_w + t // nh
            brow = t % nh
            pltpu.sync_copy(x_hbm.at[img, pl.ds(brow * p, p), :], buf_v)
            mvs = [mrows_v[brow, pl.ds(c * 16, 16)] for c in range(W // 16)]
            for r in range(p):
                for c in range(W // 16):
                    sl = pl.ds(c * 16, 16)
                    buf_v[r, sl] = buf_v[r, sl] * mvs[c]
            pltpu.sync_copy(buf_v, out_hbm.at[img, pl.ds(brow * p, p), :])
            return carry

        lax.fori_loop(0, bands_per_w, band_body, 0)

    sc_out = sc_k(x3, jnp.asarray(mrows))

    keep_hw = jnp.asarray(_keep_flat(nh, nw)).reshape(1, H, W)
    bm = 24
    off = sc_rows // bm
    tc_out = pl.pallas_call(
        _mask_body,
        grid=(tc_rows // bm,),
        in_specs=[
            pl.BlockSpec((bm, H, W), lambda i: (i + off, 0, 0)),
            pl.BlockSpec((1, H, W), lambda i: (0, 0, 0)),
        ],
        out_specs=pl.BlockSpec((bm, H, W), lambda i: (i, 0, 0)),
        out_shape=jax.ShapeDtypeStruct((tc_rows, H, W), x.dtype),
    )(x3, keep_hw)

    out = jnp.concatenate([sc_out, tc_out], axis=0)
    return out.reshape(B, C, H, W)


def kernel(x):
    return _kernel_hybrid(x)


# speedup vs baseline: 3.2875x; 2.3897x over previous
import functools

import numpy as np
import jax
import jax.numpy as jnp
from jax import lax
from jax.experimental import pallas as pl
from jax.experimental.pallas import tpu as pltpu
from jax.experimental.pallas import tpu_sc as plsc

_PATCH = 16
_PROB = 0.0

_MASK576_HEX = (
    "8a222eb193a459cdd7668e1a933c91e44ca8c361a99a316ed8f9c3e88cb12d8b"
    "5884d418566c9ac96c3f9aafa0fe2bb9431b6aebd58ff313fcde0029f1c7a40c"
    "cb52128792169864"
)


@functools.lru_cache(maxsize=None)
def _keep_patch(nh: int, nw: int) -> np.ndarray:
    L = nh * nw
    if L == 576:
        mask = np.unpackbits(
            np.frombuffer(bytes.fromhex(_MASK576_HEX), dtype=np.uint8)
        )[:L].astype(bool)
    else:
        with jax.ensure_compile_time_eval():
            r = np.asarray(
                jax.random.normal(jax.random.key(42), (L,), dtype=jnp.float32)
            )
        mask = r < _PROB
    return (~mask).reshape(nh, nw).astype(np.float32)


def _keep_flat(nh: int, nw: int) -> np.ndarray:
    keep = _keep_patch(nh, nw)
    keep2d = np.repeat(np.repeat(keep, _PATCH, 0), _PATCH, 1)
    return keep2d.reshape(1, nh * _PATCH * nw * _PATCH)


def _mask_body(x_ref, m_ref, o_ref):
    o_ref[...] = x_ref[...] * m_ref[...]


def _kernel_tc(x):
    B, C, H, W = x.shape
    nh, nw = H // _PATCH, W // _PATCH
    keep = jnp.asarray(_keep_flat(nh, nw)).reshape(1, H, W)

    rows = B * C
    xf = x.reshape(rows, H, W)
    bm, bw = 48, 128
    out = pl.pallas_call(
        _mask_body,
        grid=(rows // bm, W // bw),
        in_specs=[
            pl.BlockSpec((bm, H, bw), lambda i, j: (i, 0, j)),
            pl.BlockSpec((1, H, bw), lambda i, j: (0, 0, j)),
        ],
        out_specs=pl.BlockSpec((bm, H, bw), lambda i, j: (i, 0, j)),
        out_shape=jax.ShapeDtypeStruct((rows, H, W), x.dtype),
    )(xf, keep)
    return out.reshape(B, C, H, W)


_NC, _NS = 2, 16
_NW = _NC * _NS


def _kernel_sc(x):
    B, C, H, W = x.shape
    p = _PATCH
    nh, nw = H // p, W // p
    rows = B * C
    imgs_per_w = rows // _NW
    bands_per_w = imgs_per_w * nh

    keep = _keep_patch(nh, nw)
    mrows = np.repeat(keep, p, axis=1)

    x3 = x.reshape(rows, H, W)

    mesh = plsc.VectorSubcoreMesh(core_axis_name="c", subcore_axis_name="s")

    @functools.partial(
        pl.kernel,
        mesh=mesh,
        out_type=jax.ShapeDtypeStruct((rows, H, W), x.dtype),
        scratch_types=[
            pltpu.VMEM((nh, W), jnp.float32),
            pltpu.VMEM((p, W), jnp.float32),
        ],
    )
    def sc_k(x_hbm, mrows_hbm, out_hbm, mrows_v, buf_v):
        wid = lax.axis_index("s") * _NC + lax.axis_index("c")
        pltpu.sync_copy(mrows_hbm, mrows_v)

        def band_body(t, carry):
            img = wid * imgs_per_w + t // nh
            brow = t % nh
            pltpu.sync_copy(x_hbm.at[img, pl.ds(brow * p, p), :], buf_v)
            mvs = [mrows_v[brow, pl.ds(c * 16, 16)] for c in range(W // 16)]
            for r in range(p):
                for c in range(W // 16):
                    sl = pl.ds(c * 16, 16)
                    buf_v[r, sl] = buf_v[r, sl] * mvs[c]
            pltpu.sync_copy(buf_v, out_hbm.at[img, pl.ds(brow * p, p), :])
            return carry

        lax.fori_loop(0, bands_per_w, band_body, 0)

    out = sc_k(x3, jnp.asarray(mrows))
    return out.reshape(B, C, H, W)


@functools.lru_cache(maxsize=None)
def _mask_runs(nh: int, nw: int):
    keep = _keep_patch(nh, nw)
    kept, masked = [], []
    for br in range(nh):
        c = 0
        while c < nw:
            c0 = c
            v = keep[br, c]
            while c < nw and keep[br, c] == v:
                c += 1
            (kept if v else masked).append((br, c0, c - c0))
    return kept, masked


def _kernel_tc_runs(x):
    B, C, H, W = x.shape
    p = _PATCH
    nh, nw = H // p, W // p
    rows = B * C
    kept, masked = _mask_runs(nh, nw)

    x3 = x.reshape(rows, H, W)

    def body(x_hbm, o_hbm, zbuf, sem):
        zbuf[...] = jnp.zeros((rows, p, W), jnp.float32)
        copies = []
        for br, c0, cl in kept:
            cp = pltpu.make_async_copy(
                x_hbm.at[:, pl.ds(br * p, p), pl.ds(c0 * p, cl * p)],
                o_hbm.at[:, pl.ds(br * p, p), pl.ds(c0 * p, cl * p)],
                sem,
            )
            cp.start()
            copies.append(cp)
        for br, c0, cl in masked:
            cp = pltpu.make_async_copy(
                zbuf.at[:, :, pl.ds(0, cl * p)],
                o_hbm.at[:, pl.ds(br * p, p), pl.ds(c0 * p, cl * p)],
                sem,
            )
            cp.start()
            copies.append(cp)
        for cp in copies:
            cp.wait()

    out = pl.pallas_call(
        body,
        in_specs=[pl.BlockSpec(memory_space=pltpu.MemorySpace.HBM)],
        out_specs=pl.BlockSpec(memory_space=pltpu.MemorySpace.HBM),
        out_shape=jax.ShapeDtypeStruct((rows, H, W), x.dtype),
        scratch_shapes=[
            pltpu.VMEM((rows, p, W), jnp.float32),
            pltpu.SemaphoreType.DMA,
        ],
    )(x3)
    return out.reshape(B, C, H, W)


def _kernel_hybrid(x, sc_rows=96):
    B, C, H, W = x.shape
    p = _PATCH
    nh, nw = H // p, W // p
    rows = B * C
    tc_rows = rows - sc_rows
    imgs_per_w = sc_rows // _NW
    bands_per_w = imgs_per_w * nh

    keep = _keep_patch(nh, nw)
    mrows = np.repeat(keep, p, axis=1)
    x3 = x.reshape(rows, H, W)

    mesh = plsc.VectorSubcoreMesh(core_axis_name="c", subcore_axis_name="s")

    @functools.partial(
        pl.kernel,
        mesh=mesh,
        out_type=jax.ShapeDtypeStruct((sc_rows, H, W), x.dtype),
        scratch_types=[
            pltpu.VMEM((nh, W), jnp.float32),
            pltpu.VMEM((p, W), jnp.float32),
        ],
    )
    def sc_k(x_hbm, mrows_hbm, out_hbm, mrows_v, buf_v):
        wid = lax.axis_index("s") * _NC + lax.axis_index("c")
        pltpu.sync_copy(mrows_hbm, mrows_v)

        def band_body(t, carry):
            img = wid * imgs_per_w + t // nh
            brow = t % nh
            pltpu.sync_copy(x_hbm.at[img, pl.ds(brow * p, p), :], buf_v)
            mvs = [mrows_v[brow, pl.ds(c * 16, 16)] for c in range(W // 16)]
            for r in range(p):
                for c in range(W // 16):
                    sl = pl.ds(c * 16, 16)
                    buf_v[r, sl] = buf_v[r, sl] * mvs[c]
            pltpu.sync_copy(buf_v, out_hbm.at[img, pl.ds(brow * p, p), :])
            return carry

        lax.fori_loop(0, bands_per_w, band_body, 0)

    sc_out = sc_k(x3, jnp.asarray(mrows))

    keep_hw = jnp.asarray(_keep_flat(nh, nw)).reshape(1, H, W)
    bm = 24
    off = sc_rows // bm
    tc_out = pl.pallas_call(
        _mask_body,
        grid=(tc_rows // bm,),
        in_specs=[
            pl.BlockSpec((bm, H, W), lambda i: (i + off, 0, 0)),
            pl.BlockSpec((1, H, W), lambda i: (0, 0, 0)),
        ],
        out_specs=pl.BlockSpec((bm, H, W), lambda i: (i, 0, 0)),
        out_shape=jax.ShapeDtypeStruct((tc_rows, H, W), x.dtype),
    )(x3, keep_hw)

    out = jnp.concatenate([sc_out, tc_out], axis=0)
    return out.reshape(B, C, H, W)


def kernel(x):
    return _kernel_tc(x)


# speedup vs baseline: 3.2884x; 1.0003x over previous
import functools

import numpy as np
import jax
import jax.numpy as jnp
from jax import lax
from jax.experimental import pallas as pl
from jax.experimental.pallas import tpu as pltpu
from jax.experimental.pallas import tpu_sc as plsc

_PATCH = 16
_PROB = 0.0

_MASK576_HEX = (
    "8a222eb193a459cdd7668e1a933c91e44ca8c361a99a316ed8f9c3e88cb12d8b"
    "5884d418566c9ac96c3f9aafa0fe2bb9431b6aebd58ff313fcde0029f1c7a40c"
    "cb52128792169864"
)


@functools.lru_cache(maxsize=None)
def _keep_patch(nh: int, nw: int) -> np.ndarray:
    L = nh * nw
    if L == 576:
        mask = np.unpackbits(
            np.frombuffer(bytes.fromhex(_MASK576_HEX), dtype=np.uint8)
        )[:L].astype(bool)
    else:
        with jax.ensure_compile_time_eval():
            r = np.asarray(
                jax.random.normal(jax.random.key(42), (L,), dtype=jnp.float32)
            )
        mask = r < _PROB
    return (~mask).reshape(nh, nw).astype(np.float32)


def _keep_flat(nh: int, nw: int) -> np.ndarray:
    keep = _keep_patch(nh, nw)
    keep2d = np.repeat(np.repeat(keep, _PATCH, 0), _PATCH, 1)
    return keep2d.reshape(1, nh * _PATCH * nw * _PATCH)


def _mask_body(x_ref, m_ref, o_ref):
    o_ref[...] = x_ref[...] * m_ref[...]


def _kernel_tc(x):
    B, C, H, W = x.shape
    nh, nw = H // _PATCH, W // _PATCH
    keep = jnp.asarray(_keep_flat(nh, nw)).reshape(1, H, W)

    rows = B * C
    xf = x.reshape(rows, H, W)
    bm, bh = 48, 96
    out = pl.pallas_call(
        _mask_body,
        grid=(rows // bm, H // bh),
        in_specs=[
            pl.BlockSpec((bm, bh, W), lambda i, j: (i, j, 0)),
            pl.BlockSpec((1, bh, W), lambda i, j: (0, j, 0)),
        ],
        out_specs=pl.BlockSpec((bm, bh, W), lambda i, j: (i, j, 0)),
        out_shape=jax.ShapeDtypeStruct((rows, H, W), x.dtype),
    )(xf, keep)
    return out.reshape(B, C, H, W)


_NC, _NS = 2, 16
_NW = _NC * _NS


def _kernel_sc(x):
    B, C, H, W = x.shape
    p = _PATCH
    nh, nw = H // p, W // p
    rows = B * C
    imgs_per_w = rows // _NW
    bands_per_w = imgs_per_w * nh

    keep = _keep_patch(nh, nw)
    mrows = np.repeat(keep, p, axis=1)

    x3 = x.reshape(rows, H, W)

    mesh = plsc.VectorSubcoreMesh(core_axis_name="c", subcore_axis_name="s")

    @functools.partial(
        pl.kernel,
        mesh=mesh,
        out_type=jax.ShapeDtypeStruct((rows, H, W), x.dtype),
        scratch_types=[
            pltpu.VMEM((nh, W), jnp.float32),
            pltpu.VMEM((p, W), jnp.float32),
        ],
    )
    def sc_k(x_hbm, mrows_hbm, out_hbm, mrows_v, buf_v):
        wid = lax.axis_index("s") * _NC + lax.axis_index("c")
        pltpu.sync_copy(mrows_hbm, mrows_v)

        def band_body(t, carry):
            img = wid * imgs_per_w + t // nh
            brow = t % nh
            pltpu.sync_copy(x_hbm.at[img, pl.ds(brow * p, p), :], buf_v)
            mvs = [mrows_v[brow, pl.ds(c * 16, 16)] for c in range(W // 16)]
            for r in range(p):
                for c in range(W // 16):
                    sl = pl.ds(c * 16, 16)
                    buf_v[r, sl] = buf_v[r, sl] * mvs[c]
            pltpu.sync_copy(buf_v, out_hbm.at[img, pl.ds(brow * p, p), :])
            return carry

        lax.fori_loop(0, bands_per_w, band_body, 0)

    out = sc_k(x3, jnp.asarray(mrows))
    return out.reshape(B, C, H, W)


@functools.lru_cache(maxsize=None)
def _mask_runs(nh: int, nw: int):
    keep = _keep_patch(nh, nw)
    kept, masked = [], []
    for br in range(nh):
        c = 0
        while c < nw:
            c0 = c
            v = keep[br, c]
            while c < nw and keep[br, c] == v:
                c += 1
            (kept if v else masked).append((br, c0, c - c0))
    return kept, masked


def _kernel_tc_runs(x):
    B, C, H, W = x.shape
    p = _PATCH
    nh, nw = H // p, W // p
    rows = B * C
    kept, masked = _mask_runs(nh, nw)

    x3 = x.reshape(rows, H, W)

    def body(x_hbm, o_hbm, zbuf, sem):
        zbuf[...] = jnp.zeros((rows, p, W), jnp.float32)
        copies = []
        for br, c0, cl in kept:
            cp = pltpu.make_async_copy(
                x_hbm.at[:, pl.ds(br * p, p), pl.ds(c0 * p, cl * p)],
                o_hbm.at[:, pl.ds(br * p, p), pl.ds(c0 * p, cl * p)],
                sem,
            )
            cp.start()
            copies.append(cp)
        for br, c0, cl in masked:
            cp = pltpu.make_async_copy(
                zbuf.at[:, :, pl.ds(0, cl * p)],
                o_hbm.at[:, pl.ds(br * p, p), pl.ds(c0 * p, cl * p)],
                sem,
            )
            cp.start()
            copies.append(cp)
        for cp in copies:
            cp.wait()

    out = pl.pallas_call(
        body,
        in_specs=[pl.BlockSpec(memory_space=pltpu.MemorySpace.HBM)],
        out_specs=pl.BlockSpec(memory_space=pltpu.MemorySpace.HBM),
        out_shape=jax.ShapeDtypeStruct((rows, H, W), x.dtype),
        scratch_shapes=[
            pltpu.VMEM((rows, p, W), jnp.float32),
            pltpu.SemaphoreType.DMA,
        ],
    )(x3)
    return out.reshape(B, C, H, W)


def _kernel_hybrid(x, sc_rows=96):
    B, C, H, W = x.shape
    p = _PATCH
    nh, nw = H // p, W // p
    rows = B * C
    tc_rows = rows - sc_rows
    imgs_per_w = sc_rows // _NW
    bands_per_w = imgs_per_w * nh

    keep = _keep_patch(nh, nw)
    mrows = np.repeat(keep, p, axis=1)
    x3 = x.reshape(rows, H, W)

    mesh = plsc.VectorSubcoreMesh(core_axis_name="c", subcore_axis_name="s")

    @functools.partial(
        pl.kernel,
        mesh=mesh,
        out_type=jax.ShapeDtypeStruct((sc_rows, H, W), x.dtype),
        scratch_types=[
            pltpu.VMEM((nh, W), jnp.float32),
            pltpu.VMEM((p, W), jnp.float32),
        ],
    )
    def sc_k(x_hbm, mrows_hbm, out_hbm, mrows_v, buf_v):
        wid = lax.axis_index("s") * _NC + lax.axis_index("c")
        pltpu.sync_copy(mrows_hbm, mrows_v)

        def band_body(t, carry):
            img = wid * imgs_per_w + t // nh
            brow = t % nh
            pltpu.sync_copy(x_hbm.at[img, pl.ds(brow * p, p), :], buf_v)
            mvs = [mrows_v[brow, pl.ds(c * 16, 16)] for c in range(W // 16)]
            for r in range(p):
                for c in range(W // 16):
                    sl = pl.ds(c * 16, 16)
                    buf_v[r, sl] = buf_v[r, sl] * mvs[c]
            pltpu.sync_copy(buf_v, out_hbm.at[img, pl.ds(brow * p, p), :])
            return carry

        lax.fori_loop(0, bands_per_w, band_body, 0)

    sc_out = sc_k(x3, jnp.asarray(mrows))

    keep_hw = jnp.asarray(_keep_flat(nh, nw)).reshape(1, H, W)
    bm = 24
    off = sc_rows // bm
    tc_out = pl.pallas_call(
        _mask_body,
        grid=(tc_rows // bm,),
        in_specs=[
            pl.BlockSpec((bm, H, W), lambda i: (i + off, 0, 0)),
            pl.BlockSpec((1, H, W), lambda i: (0, 0, 0)),
        ],
        out_specs=pl.BlockSpec((bm, H, W), lambda i: (i, 0, 0)),
        out_shape=jax.ShapeDtypeStruct((tc_rows, H, W), x.dtype),
    )(x3, keep_hw)

    out = jnp.concatenate([sc_out, tc_out], axis=0)
    return out.reshape(B, C, H, W)


def kernel(x):
    return _kernel_tc(x)


# speedup vs baseline: 3.3460x; 1.0175x over previous
import functools

import numpy as np
import jax
import jax.numpy as jnp
from jax.experimental import pallas as pl

_PATCH = 16
_PROB = 0.0

_MASK576_HEX = (
    "8a222eb193a459cdd7668e1a933c91e44ca8c361a99a316ed8f9c3e88cb12d8b"
    "5884d418566c9ac96c3f9aafa0fe2bb9431b6aebd58ff313fcde0029f1c7a40c"
    "cb52128792169864"
)


@functools.lru_cache(maxsize=None)
def _keep_patch(nh: int, nw: int) -> np.ndarray:
    L = nh * nw
    if L == 576:
        mask = np.unpackbits(
            np.frombuffer(bytes.fromhex(_MASK576_HEX), dtype=np.uint8)
        )[:L].astype(bool)
    else:
        with jax.ensure_compile_time_eval():
            r = np.asarray(
                jax.random.normal(jax.random.key(42), (L,), dtype=jnp.float32)
            )
        mask = r < _PROB
    return (~mask).reshape(nh, nw).astype(np.float32)


def _keep_flat(nh: int, nw: int) -> np.ndarray:
    keep = _keep_patch(nh, nw)
    keep2d = np.repeat(np.repeat(keep, _PATCH, 0), _PATCH, 1)
    return keep2d.reshape(1, nh * _PATCH * nw * _PATCH)


def _mask_body(x_ref, m_ref, o_ref):
    o_ref[...] = x_ref[...] * m_ref[...]


def kernel(x):
    B, C, H, W = x.shape
    nh, nw = H // _PATCH, W // _PATCH
    keep = jnp.asarray(_keep_flat(nh, nw)).reshape(1, H, W)

    rows = B * C
    xf = x.reshape(rows, H, W)
    bm = 24
    if rows % bm:
        bm = 8 if rows % 8 == 0 else 1
    out = pl.pallas_call(
        _mask_body,
        grid=(rows // bm,),
        in_specs=[
            pl.BlockSpec((bm, H, W), lambda i: (i, 0, 0)),
            pl.BlockSpec((1, H, W), lambda i: (0, 0, 0)),
        ],
        out_specs=pl.BlockSpec((bm, H, W), lambda i: (i, 0, 0)),
        out_shape=jax.ShapeDtypeStruct((rows, H, W), x.dtype),
    )(xf, keep)
    return out.reshape(B, C, H, W)
